# baseline (speedup 1.0000x reference)
import functools

import jax
import jax.numpy as jnp
from jax import lax
from jax.experimental import pallas as pl

PL = 48
NC = 16
DIM = 3
OUT_CH = 13


def _ptconv_body(zhi_ref, zlo_ref, ptst_ref, q_ref, cen_ref,
                 w1_ref, b1_ref, w2_ref, b2_ref, w3_ref, b3_ref, wm_ref,
                 out_ref, *, K, N, C, Mt, cout, T):
    z_hi = zhi_ref[0]
    z_lo = zlo_ref[0]
    pts_t = ptst_ref[0]
    cen = cen_ref[...]
    w1 = w1_ref[...]
    b1 = b1_ref[...]
    w2 = w2_ref[...]
    b2 = b2_ref[...]
    w3 = w3_ref[...]
    b3 = b3_ref[...]
    wm = wm_ref[...]

    def tile(t, carry):
        q = q_ref[0, pl.ds(t * Mt, Mt), :]

        d2 = None
        for d in range(DIM):
            diff = q[:, d:d + 1] - pts_t[d:d + 1, :]
            sq = diff * diff
            d2 = sq if d2 is None else d2 + sq

        ohs = []
        d2w = d2
        for _ in range(K):
            mn = jnp.min(d2w, axis=1, keepdims=True)
            hot = d2w == mn
            ohs.append(hot.astype(jnp.bfloat16))
            d2w = jnp.where(hot, jnp.float32(jnp.inf), d2w)

        oh = jnp.concatenate(ohs, axis=0)
        g = (jnp.dot(oh, z_hi, preferred_element_type=jnp.float32) +
             jnp.dot(oh, z_lo, preferred_element_type=jnp.float32))
        fks = [g[k * Mt:(k + 1) * Mt, :C] for k in range(K)]
        prels = [g[k * Mt:(k + 1) * Mt, C:C + DIM] - q for k in range(K)]

        maxr = None
        for pr in prels:
            r2 = jnp.sum(pr * pr, axis=1, keepdims=True)
            maxr = r2 if maxr is None else jnp.maximum(maxr, r2)
        maxi = jnp.sqrt(maxr)
        maxi = jnp.where(maxi == 0.0, 1.0, maxi)
        inv = 1.0 / maxi

        pn_all = jnp.concatenate(prels, axis=0) * jnp.concatenate(
            [inv] * K, axis=0)
        dm = jnp.concatenate(
            [pn_all[:, d:d + 1] - cen[d:d + 1, :] for d in range(DIM)],
            axis=1)
        h = jnp.maximum(
            jnp.dot(dm, w1, preferred_element_type=jnp.float32) + b1, 0.0)
        h = jnp.maximum(
            jnp.dot(h, w2, preferred_element_type=jnp.float32) + b2, 0.0)
        dv = jnp.maximum(
            jnp.dot(h, w3, preferred_element_type=jnp.float32) + b3, 0.0)

        fs_n = [None] * NC
        for k in range(K):
            dvk = dv[k * Mt:(k + 1) * Mt]
            for n in range(NC):
                t2 = fks[k] * dvk[:, n:n + 1]
                fs_n[n] = t2 if fs_n[n] is None else fs_n[n] + t2
        fs = jnp.concatenate(fs_n, axis=1)

        out = jnp.dot(fs, wm, preferred_element_type=jnp.float32)
        out_ref[0, pl.ds(t * Mt, Mt), :] = out * (1.0 / K)
        return carry

    lax.fori_loop(0, T, tile, 0, unroll=False)


def _ptconv(feat, points, queries, K, p, Mt):
    B, N, C = feat.shape
    M = queries.shape[1]
    cout = p["weight"].shape[2]
    points_t = jnp.transpose(points, (0, 2, 1))
    z = jnp.concatenate([feat, points], axis=2)
    z_hi = z.astype(jnp.bfloat16)
    z_lo = (z - z_hi.astype(jnp.float32)).astype(jnp.bfloat16)
    wmat = jnp.transpose(p["weight"], (1, 0, 2)).reshape(NC * C, cout)
    body = functools.partial(_ptconv_body, K=K, N=N, C=C, Mt=Mt, cout=cout,
                             T=M // Mt)
    grid = (B,)
    full2d = lambda shape: pl.BlockSpec(shape, lambda b: (0, 0))
    out = pl.pallas_call(
        body,
        grid=grid,
        in_specs=[
            pl.BlockSpec((1, N, C + DIM), lambda b: (b, 0, 0)),
            pl.BlockSpec((1, N, C + DIM), lambda b: (b, 0, 0)),
            pl.BlockSpec((1, DIM, N), lambda b: (b, 0, 0)),
            pl.BlockSpec((1, M, DIM), lambda b: (b, 0, 0)),
            full2d((DIM, NC)),
            full2d((DIM * NC, 2 * NC)),
            full2d((1, 2 * NC)),
            full2d((2 * NC, NC)),
            full2d((1, NC)),
            full2d((NC, NC)),
            full2d((1, NC)),
            full2d((NC * C, cout)),
        ],
        out_specs=pl.BlockSpec((1, M, cout), lambda b: (b, 0, 0)),
        out_shape=jax.ShapeDtypeStruct((B, M, cout), jnp.float32),
    )(z_hi, z_lo, points_t, queries, p["centers"], p["l1w"].T,
      p["l1b"].reshape(1, -1), p["l2w"].T, p["l2b"].reshape(1, -1),
      p["l3w"].T, p["l3b"].reshape(1, -1), wmat)
    return out


def _bn_relu_body(x_ref, g_ref, b_ref, out_ref):
    x = x_ref[...]
    m = jnp.mean(x, axis=0, keepdims=True)
    xc = x - m
    v = jnp.mean(xc * xc, axis=0, keepdims=True)
    y = xc / jnp.sqrt(v + 1e-5) * g_ref[...] + b_ref[...]
    out_ref[...] = jnp.maximum(y, 0.0)


def _bn_relu(h, g, b):
    B, M, C = h.shape
    x = h.reshape(B * M, C)
    out = pl.pallas_call(
        _bn_relu_body,
        out_shape=jax.ShapeDtypeStruct((B * M, C), jnp.float32),
    )(x, g.reshape(1, -1), b.reshape(1, -1))
    return out.reshape(B, M, C)


def _bn_stats_body(x_ref, m_ref, v_ref):
    x = x_ref[...]
    m = jnp.mean(x, axis=0, keepdims=True)
    xc = x - m
    m_ref[...] = m
    v_ref[...] = jnp.mean(xc * xc, axis=0, keepdims=True)


def _heads_body(x_ref, m_ref, v_ref, g_ref, b_ref, fw_ref, fb_ref, cw_ref,
                cb_ref, xout_ref, cout_ref):
    xc = x_ref[...] - m_ref[...]
    y = jnp.maximum(
        xc / jnp.sqrt(v_ref[...] + 1e-5) * g_ref[...] + b_ref[...], 0.0)
    xout_ref[...] = (
        jnp.dot(y, fw_ref[...], preferred_element_type=jnp.float32) +
        fb_ref[...])
    cout_ref[...] = (
        jnp.dot(y, cw_ref[...], preferred_element_type=jnp.float32) +
        cb_ref[...])


def _bn_relu_heads(h, g, b, fw, fb, cw, cb):
    B, M, C = h.shape
    R = B * M
    x = h.reshape(R, C)
    mean, var = pl.pallas_call(
        _bn_stats_body,
        out_shape=(
            jax.ShapeDtypeStruct((1, C), jnp.float32),
            jax.ShapeDtypeStruct((1, C), jnp.float32),
        ),
    )(x)
    Rt = 4096
    full2d = lambda shape: pl.BlockSpec(shape, lambda t: (0, 0))
    xout, cout = pl.pallas_call(
        _heads_body,
        grid=(R // Rt,),
        in_specs=[
            pl.BlockSpec((Rt, C), lambda t: (t, 0)),
            full2d((1, C)),
            full2d((1, C)),
            full2d((1, C)),
            full2d((1, C)),
            full2d((C, OUT_CH)),
            full2d((1, OUT_CH)),
            full2d((C, 1)),
            full2d((1, 1)),
        ],
        out_specs=(
            pl.BlockSpec((Rt, OUT_CH), lambda t: (t, 0)),
            pl.BlockSpec((Rt, 1), lambda t: (t, 0)),
        ),
        out_shape=(
            jax.ShapeDtypeStruct((R, OUT_CH), jnp.float32),
            jax.ShapeDtypeStruct((R, 1), jnp.float32),
        ),
    )(x, mean, var, g.reshape(1, -1), b.reshape(1, -1), fw.T,
      fb.reshape(1, -1), cw.T, cb.reshape(1, -1))
    return xout.reshape(B, M, OUT_CH), cout.reshape(B, M, 1)


def kernel(x, input_pts, x6, pts6, x5, pts5, x4, pts4, x3, pts3, x2, pts2,
           params):
    p = params
    h = _ptconv(x6, pts6, pts5, 4, p["cv5d"], Mt=64)
    h = _bn_relu(h, p["bn5d_g"], p["bn5d_b"])
    h = jnp.concatenate([h, x5], axis=2)

    h = _ptconv(h, pts5, pts4, 4, p["cv4d"], Mt=256)
    h = _bn_relu(h, p["bn4d_g"], p["bn4d_b"])
    h = jnp.concatenate([h, x4], axis=2)

    h = _ptconv(h, pts4, pts3, 4, p["cv3d"], Mt=512)
    h = _bn_relu(h, p["bn3d_g"], p["bn3d_b"])
    h = jnp.concatenate([h, x3], axis=2)

    h = _ptconv(h, pts3, pts2, 8, p["cv2d"], Mt=512)
    h = _bn_relu(h, p["bn2d_g"], p["bn2d_b"])
    h = jnp.concatenate([h, x2], axis=2)

    h = _ptconv(h, pts2, input_pts, 8, p["cv1d"], Mt=512)
    xout, cout = _bn_relu_heads(h, p["bn1d_g"], p["bn1d_b"], p["fcout_w"],
                                p["fcout_b"], p["ccout_w"], p["ccout_b"])
    return (xout, cout)


# speedup vs baseline: 1.3564x; 1.3564x over previous
import functools

import jax
import jax.numpy as jnp
from jax import lax
from jax.experimental import pallas as pl

PL = 48
NC = 16
DIM = 3
OUT_CH = 13


def _ptconv_body(zhi_ref, zlo_ref, ptst_ref, q_ref, cen_ref,
                 w1_ref, b1_ref, w2_ref, b2_ref, w3_ref, b3_ref, wm_ref,
                 out_ref, *, K, N, C, Mt, cout, T):
    z_hi = zhi_ref[0]
    z_lo = zlo_ref[0]
    pts_t = ptst_ref[0]
    cen = cen_ref[...]
    w1 = w1_ref[...]
    b1 = b1_ref[...]
    w2 = w2_ref[...]
    b2 = b2_ref[...]
    w3 = w3_ref[...]
    b3 = b3_ref[...]
    wm = wm_ref[...]

    def tile(t, carry):
        q = q_ref[0, pl.ds(t * Mt, Mt), :]

        d2 = None
        for d in range(DIM):
            diff = q[:, d:d + 1] - pts_t[d:d + 1, :]
            sq = diff * diff
            d2 = sq if d2 is None else d2 + sq

        ohs = []
        d2w = d2
        for _ in range(K):
            mn = jnp.min(d2w, axis=1, keepdims=True)
            hot = d2w == mn
            ohs.append(hot.astype(jnp.bfloat16))
            d2w = jnp.where(hot, jnp.float32(jnp.inf), d2w)

        oh = jnp.concatenate(ohs, axis=0)
        g = (jnp.dot(oh, z_hi, preferred_element_type=jnp.float32) +
             jnp.dot(oh, z_lo, preferred_element_type=jnp.float32))
        fks = [g[k * Mt:(k + 1) * Mt, :C] for k in range(K)]
        prels = [g[k * Mt:(k + 1) * Mt, C:C + DIM] - q for k in range(K)]

        maxr = None
        for pr in prels:
            r2 = jnp.sum(pr * pr, axis=1, keepdims=True)
            maxr = r2 if maxr is None else jnp.maximum(maxr, r2)
        maxi = jnp.sqrt(maxr)
        maxi = jnp.where(maxi == 0.0, 1.0, maxi)
        inv = 1.0 / maxi

        pn_all = jnp.concatenate(prels, axis=0) * jnp.concatenate(
            [inv] * K, axis=0)
        dm = jnp.concatenate(
            [pn_all[:, d:d + 1] - cen[d:d + 1, :] for d in range(DIM)],
            axis=1)
        h = jnp.maximum(
            jnp.dot(dm, w1, preferred_element_type=jnp.float32) + b1, 0.0)
        h = jnp.maximum(
            jnp.dot(h, w2, preferred_element_type=jnp.float32) + b2, 0.0)
        dv = jnp.maximum(
            jnp.dot(h, w3, preferred_element_type=jnp.float32) + b3, 0.0)

        fs_n = []
        for n in range(NC):
            acc = None
            for k in range(K):
                t2 = fks[k] * dv[k * Mt:(k + 1) * Mt, n:n + 1]
                acc = t2 if acc is None else acc + t2
            fs_n.append(acc)
        fs = jnp.concatenate(fs_n, axis=1)

        out = jnp.dot(fs, wm, preferred_element_type=jnp.float32)
        out_ref[0, pl.ds(t * Mt, Mt), :] = out * (1.0 / K)
        return carry

    lax.fori_loop(0, T, tile, 0, unroll=False)


def _ptconv(feat, points, queries, K, p, Mt):
    B, N, C = feat.shape
    M = queries.shape[1]
    cout = p["weight"].shape[2]
    points_t = jnp.transpose(points, (0, 2, 1))
    z = jnp.concatenate([feat, points], axis=2)
    z_hi = z.astype(jnp.bfloat16)
    z_lo = (z - z_hi.astype(jnp.float32)).astype(jnp.bfloat16)
    wmat = jnp.transpose(p["weight"], (1, 0, 2)).reshape(NC * C, cout)
    body = functools.partial(_ptconv_body, K=K, N=N, C=C, Mt=Mt, cout=cout,
                             T=M // Mt)
    grid = (B,)
    full2d = lambda shape: pl.BlockSpec(shape, lambda b: (0, 0))
    out = pl.pallas_call(
        body,
        grid=grid,
        in_specs=[
            pl.BlockSpec((1, N, C + DIM), lambda b: (b, 0, 0)),
            pl.BlockSpec((1, N, C + DIM), lambda b: (b, 0, 0)),
            pl.BlockSpec((1, DIM, N), lambda b: (b, 0, 0)),
            pl.BlockSpec((1, M, DIM), lambda b: (b, 0, 0)),
            full2d((DIM, NC)),
            full2d((DIM * NC, 2 * NC)),
            full2d((1, 2 * NC)),
            full2d((2 * NC, NC)),
            full2d((1, NC)),
            full2d((NC, NC)),
            full2d((1, NC)),
            full2d((NC * C, cout)),
        ],
        out_specs=pl.BlockSpec((1, M, cout), lambda b: (b, 0, 0)),
        out_shape=jax.ShapeDtypeStruct((B, M, cout), jnp.float32),
    )(z_hi, z_lo, points_t, queries, p["centers"], p["l1w"].T,
      p["l1b"].reshape(1, -1), p["l2w"].T, p["l2b"].reshape(1, -1),
      p["l3w"].T, p["l3b"].reshape(1, -1), wmat)
    return out


def _bn_relu_body(x_ref, g_ref, b_ref, out_ref):
    x = x_ref[...]
    m = jnp.mean(x, axis=0, keepdims=True)
    xc = x - m
    v = jnp.mean(xc * xc, axis=0, keepdims=True)
    y = xc / jnp.sqrt(v + 1e-5) * g_ref[...] + b_ref[...]
    out_ref[...] = jnp.maximum(y, 0.0)


def _bn_relu(h, g, b):
    B, M, C = h.shape
    x = h.reshape(B * M, C)
    out = pl.pallas_call(
        _bn_relu_body,
        out_shape=jax.ShapeDtypeStruct((B * M, C), jnp.float32),
    )(x, g.reshape(1, -1), b.reshape(1, -1))
    return out.reshape(B, M, C)


def _bn_stats_body(x_ref, m_ref, v_ref):
    x = x_ref[...]
    m = jnp.mean(x, axis=0, keepdims=True)
    xc = x - m
    m_ref[...] = m
    v_ref[...] = jnp.mean(xc * xc, axis=0, keepdims=True)


def _heads_body(x_ref, m_ref, v_ref, g_ref, b_ref, fw_ref, fb_ref, cw_ref,
                cb_ref, xout_ref, cout_ref):
    xc = x_ref[...] - m_ref[...]
    y = jnp.maximum(
        xc / jnp.sqrt(v_ref[...] + 1e-5) * g_ref[...] + b_ref[...], 0.0)
    xout_ref[...] = (
        jnp.dot(y, fw_ref[...], preferred_element_type=jnp.float32) +
        fb_ref[...])
    cout_ref[...] = (
        jnp.dot(y, cw_ref[...], preferred_element_type=jnp.float32) +
        cb_ref[...])


def _bn_relu_heads(h, g, b, fw, fb, cw, cb):
    B, M, C = h.shape
    R = B * M
    x = h.reshape(R, C)
    mean, var = pl.pallas_call(
        _bn_stats_body,
        out_shape=(
            jax.ShapeDtypeStruct((1, C), jnp.float32),
            jax.ShapeDtypeStruct((1, C), jnp.float32),
        ),
    )(x)
    Rt = 4096
    full2d = lambda shape: pl.BlockSpec(shape, lambda t: (0, 0))
    xout, cout = pl.pallas_call(
        _heads_body,
        grid=(R // Rt,),
        in_specs=[
            pl.BlockSpec((Rt, C), lambda t: (t, 0)),
            full2d((1, C)),
            full2d((1, C)),
            full2d((1, C)),
            full2d((1, C)),
            full2d((C, OUT_CH)),
            full2d((1, OUT_CH)),
            full2d((C, 1)),
            full2d((1, 1)),
        ],
        out_specs=(
            pl.BlockSpec((Rt, OUT_CH), lambda t: (t, 0)),
            pl.BlockSpec((Rt, 1), lambda t: (t, 0)),
        ),
        out_shape=(
            jax.ShapeDtypeStruct((R, OUT_CH), jnp.float32),
            jax.ShapeDtypeStruct((R, 1), jnp.float32),
        ),
    )(x, mean, var, g.reshape(1, -1), b.reshape(1, -1), fw.T,
      fb.reshape(1, -1), cw.T, cb.reshape(1, -1))
    return xout.reshape(B, M, OUT_CH), cout.reshape(B, M, 1)


def kernel(x, input_pts, x6, pts6, x5, pts5, x4, pts4, x3, pts3, x2, pts2,
           params):
    p = params
    h = _ptconv(x6, pts6, pts5, 4, p["cv5d"], Mt=64)
    h = _bn_relu(h, p["bn5d_g"], p["bn5d_b"])
    h = jnp.concatenate([h, x5], axis=2)

    h = _ptconv(h, pts5, pts4, 4, p["cv4d"], Mt=256)
    h = _bn_relu(h, p["bn4d_g"], p["bn4d_b"])
    h = jnp.concatenate([h, x4], axis=2)

    h = _ptconv(h, pts4, pts3, 4, p["cv3d"], Mt=512)
    h = _bn_relu(h, p["bn3d_g"], p["bn3d_b"])
    h = jnp.concatenate([h, x3], axis=2)

    h = _ptconv(h, pts3, pts2, 8, p["cv2d"], Mt=512)
    h = _bn_relu(h, p["bn2d_g"], p["bn2d_b"])
    h = jnp.concatenate([h, x2], axis=2)

    h = _ptconv(h, pts2, input_pts, 8, p["cv1d"], Mt=512)
    xout, cout = _bn_relu_heads(h, p["bn1d_g"], p["bn1d_b"], p["fcout_w"],
                                p["fcout_b"], p["ccout_w"], p["ccout_b"])
    return (xout, cout)


# speedup vs baseline: 1.3646x; 1.0060x over previous
import functools

import jax
import jax.numpy as jnp
from jax import lax
from jax.experimental import pallas as pl

PL = 48
NC = 16
DIM = 3
OUT_CH = 13


def _ptconv_tile(t, q_ref, out_ref, z_hi, z_lo, pts_t, cen, w1, b1, w2, b2,
                 w3, b3, wm, *, K, N, C, Mt, cout):
    q = q_ref[0, pl.ds(t * Mt, Mt), :]

    d2 = None
    for d in range(DIM):
        diff = q[:, d:d + 1] - pts_t[d:d + 1, :]
        sq = diff * diff
        d2 = sq if d2 is None else d2 + sq

    ohs = []
    d2w = d2
    for _ in range(K):
        mn = jnp.min(d2w, axis=1, keepdims=True)
        hot = d2w == mn
        ohs.append(hot.astype(jnp.bfloat16))
        d2w = jnp.where(hot, jnp.float32(jnp.inf), d2w)

    oh = jnp.concatenate(ohs, axis=0)
    g = (jnp.dot(oh, z_hi, preferred_element_type=jnp.float32) +
         jnp.dot(oh, z_lo, preferred_element_type=jnp.float32))
    fks = [g[k * Mt:(k + 1) * Mt, :C] for k in range(K)]
    prels = [g[k * Mt:(k + 1) * Mt, C:C + DIM] - q for k in range(K)]

    maxr = None
    for pr in prels:
        r2 = jnp.sum(pr * pr, axis=1, keepdims=True)
        maxr = r2 if maxr is None else jnp.maximum(maxr, r2)
    maxi = jnp.sqrt(maxr)
    maxi = jnp.where(maxi == 0.0, 1.0, maxi)
    inv = 1.0 / maxi

    pn_all = jnp.concatenate(prels, axis=0) * jnp.concatenate(
        [inv] * K, axis=0)
    dm = jnp.concatenate(
        [pn_all[:, d:d + 1] - cen[d:d + 1, :] for d in range(DIM)],
        axis=1)
    h = jnp.maximum(
        jnp.dot(dm, w1, preferred_element_type=jnp.float32) + b1, 0.0)
    h = jnp.maximum(
        jnp.dot(h, w2, preferred_element_type=jnp.float32) + b2, 0.0)
    dv = jnp.maximum(
        jnp.dot(h, w3, preferred_element_type=jnp.float32) + b3, 0.0)

    fs_n = [None] * NC
    for k in range(K):
        dvk = dv[k * Mt:(k + 1) * Mt]
        for n in range(NC):
            t2 = fks[k] * dvk[:, n:n + 1]
            fs_n[n] = t2 if fs_n[n] is None else fs_n[n] + t2
    fs = jnp.concatenate(fs_n, axis=1)

    out = jnp.dot(fs, wm, preferred_element_type=jnp.float32)
    out_ref[0, pl.ds(t * Mt, Mt), :] = out * (1.0 / K)


def _split(z):
    z_hi = z.astype(jnp.bfloat16)
    z_lo = (z - z_hi.astype(jnp.float32)).astype(jnp.bfloat16)
    return z_hi, z_lo


def _ptconv_body(zhi_ref, zlo_ref, ptst_ref, q_ref, cen_ref,
                 w1_ref, b1_ref, w2_ref, b2_ref, w3_ref, b3_ref, wm_ref,
                 out_ref, *, K, N, C, Mt, cout, T):
    z_hi = zhi_ref[0]
    z_lo = zlo_ref[0]
    pts_t = ptst_ref[0]
    args = (z_hi, z_lo, pts_t, cen_ref[...], w1_ref[...], b1_ref[...],
            w2_ref[...], b2_ref[...], w3_ref[...], b3_ref[...], wm_ref[...])

    def tile(t, carry):
        _ptconv_tile(t, q_ref, out_ref, *args, K=K, N=N, C=C, Mt=Mt,
                     cout=cout)
        return carry

    lax.fori_loop(0, T, tile, 0, unroll=False)


def _ptconv_bn_body(hraw_ref, skip_ref, bg_ref, bb_ref, pts_ref, ptst_ref,
                    q_ref, cen_ref, w1_ref, b1_ref, w2_ref, b2_ref, w3_ref,
                    b3_ref, wm_ref, out_ref, *, K, N, C, Cp, Mt, cout, T):
    b = pl.program_id(0)
    hall = hraw_ref[...]
    m = jnp.mean(hall, axis=0, keepdims=True)
    xc = hall - m
    v = jnp.mean(xc * xc, axis=0, keepdims=True)
    hb = hraw_ref[pl.ds(b * N, N), :]
    y = jnp.maximum((hb - m) / jnp.sqrt(v + 1e-5) * bg_ref[...] + bb_ref[...],
                    0.0)
    z = jnp.concatenate([y, skip_ref[0], pts_ref[0]], axis=1)
    z_hi, z_lo = _split(z)
    pts_t = ptst_ref[0]
    args = (z_hi, z_lo, pts_t, cen_ref[...], w1_ref[...], b1_ref[...],
            w2_ref[...], b2_ref[...], w3_ref[...], b3_ref[...], wm_ref[...])

    def tile(t, carry):
        _ptconv_tile(t, q_ref, out_ref, *args, K=K, N=N, C=C, Mt=Mt,
                     cout=cout)
        return carry

    lax.fori_loop(0, T, tile, 0, unroll=False)


def _mlp_args(p):
    return (p["centers"], p["l1w"].T, p["l1b"].reshape(1, -1), p["l2w"].T,
            p["l2b"].reshape(1, -1), p["l3w"].T, p["l3b"].reshape(1, -1))


def _mlp_specs(full2d):
    return [
        full2d((DIM, NC)),
        full2d((DIM * NC, 2 * NC)),
        full2d((1, 2 * NC)),
        full2d((2 * NC, NC)),
        full2d((1, NC)),
        full2d((NC, NC)),
        full2d((1, NC)),
    ]


def _ptconv_first(feat, points, queries, K, p, Mt):
    B, N, C = feat.shape
    M = queries.shape[1]
    cout = p["weight"].shape[2]
    points_t = jnp.transpose(points, (0, 2, 1))
    z = jnp.concatenate([feat, points], axis=2)
    z_hi, z_lo = _split(z)
    wmat = jnp.transpose(p["weight"], (1, 0, 2)).reshape(NC * C, cout)
    body = functools.partial(_ptconv_body, K=K, N=N, C=C, Mt=Mt, cout=cout,
                             T=M // Mt)
    full2d = lambda shape: pl.BlockSpec(shape, lambda b: (0, 0))
    out = pl.pallas_call(
        body,
        grid=(B,),
        in_specs=[
            pl.BlockSpec((1, N, C + DIM), lambda b: (b, 0, 0)),
            pl.BlockSpec((1, N, C + DIM), lambda b: (b, 0, 0)),
            pl.BlockSpec((1, DIM, N), lambda b: (b, 0, 0)),
            pl.BlockSpec((1, M, DIM), lambda b: (b, 0, 0)),
        ] + _mlp_specs(full2d) + [full2d((NC * C, cout))],
        out_specs=pl.BlockSpec((1, M, cout), lambda b: (b, 0, 0)),
        out_shape=jax.ShapeDtypeStruct((B, M, cout), jnp.float32),
    )(z_hi, z_lo, points_t, queries, *_mlp_args(p), wmat)
    return out


def _ptconv_bn(hraw, skip, bg, bb, points, queries, K, p, Mt):
    B, N, Cp = hraw.shape
    Cs = skip.shape[2]
    C = Cp + Cs
    M = queries.shape[1]
    cout = p["weight"].shape[2]
    points_t = jnp.transpose(points, (0, 2, 1))
    wmat = jnp.transpose(p["weight"], (1, 0, 2)).reshape(NC * C, cout)
    body = functools.partial(_ptconv_bn_body, K=K, N=N, C=C, Cp=Cp, Mt=Mt,
                             cout=cout, T=M // Mt)
    full2d = lambda shape: pl.BlockSpec(shape, lambda b: (0, 0))
    out = pl.pallas_call(
        body,
        grid=(B,),
        in_specs=[
            full2d((B * N, Cp)),
            pl.BlockSpec((1, N, Cs), lambda b: (b, 0, 0)),
            full2d((1, Cp)),
            full2d((1, Cp)),
            pl.BlockSpec((1, N, DIM), lambda b: (b, 0, 0)),
            pl.BlockSpec((1, DIM, N), lambda b: (b, 0, 0)),
            pl.BlockSpec((1, M, DIM), lambda b: (b, 0, 0)),
        ] + _mlp_specs(full2d) + [full2d((NC * C, cout))],
        out_specs=pl.BlockSpec((1, M, cout), lambda b: (b, 0, 0)),
        out_shape=jax.ShapeDtypeStruct((B, M, cout), jnp.float32),
    )(hraw.reshape(B * N, Cp), skip, bg.reshape(1, -1), bb.reshape(1, -1),
      points, points_t, queries, *_mlp_args(p), wmat)
    return out


def _bn_stats_body(x_ref, m_ref, v_ref):
    x = x_ref[...]
    m = jnp.mean(x, axis=0, keepdims=True)
    xc = x - m
    m_ref[...] = m
    v_ref[...] = jnp.mean(xc * xc, axis=0, keepdims=True)


def _heads_body(x_ref, m_ref, v_ref, g_ref, b_ref, fw_ref, fb_ref, cw_ref,
                cb_ref, xout_ref, cout_ref):
    xc = x_ref[...] - m_ref[...]
    y = jnp.maximum(
        xc / jnp.sqrt(v_ref[...] + 1e-5) * g_ref[...] + b_ref[...], 0.0)
    xout_ref[...] = (
        jnp.dot(y, fw_ref[...], preferred_element_type=jnp.float32) +
        fb_ref[...])
    cout_ref[...] = (
        jnp.dot(y, cw_ref[...], preferred_element_type=jnp.float32) +
        cb_ref[...])


def _bn_relu_heads(h, g, b, fw, fb, cw, cb):
    B, M, C = h.shape
    R = B * M
    x = h.reshape(R, C)
    mean, var = pl.pallas_call(
        _bn_stats_body,
        out_shape=(
            jax.ShapeDtypeStruct((1, C), jnp.float32),
            jax.ShapeDtypeStruct((1, C), jnp.float32),
        ),
    )(x)
    Rt = 4096
    full2d = lambda shape: pl.BlockSpec(shape, lambda t: (0, 0))
    xout, cout = pl.pallas_call(
        _heads_body,
        grid=(R // Rt,),
        in_specs=[
            pl.BlockSpec((Rt, C), lambda t: (t, 0)),
            full2d((1, C)),
            full2d((1, C)),
            full2d((1, C)),
            full2d((1, C)),
            full2d((C, OUT_CH)),
            full2d((1, OUT_CH)),
            full2d((C, 1)),
            full2d((1, 1)),
        ],
        out_specs=(
            pl.BlockSpec((Rt, OUT_CH), lambda t: (t, 0)),
            pl.BlockSpec((Rt, 1), lambda t: (t, 0)),
        ),
        out_shape=(
            jax.ShapeDtypeStruct((R, OUT_CH), jnp.float32),
            jax.ShapeDtypeStruct((R, 1), jnp.float32),
        ),
    )(x, mean, var, g.reshape(1, -1), b.reshape(1, -1), fw.T,
      fb.reshape(1, -1), cw.T, cb.reshape(1, -1))
    return xout.reshape(B, M, OUT_CH), cout.reshape(B, M, 1)


def kernel(x, input_pts, x6, pts6, x5, pts5, x4, pts4, x3, pts3, x2, pts2,
           params):
    p = params
    h = _ptconv_first(x6, pts6, pts5, 4, p["cv5d"], Mt=64)
    h = _ptconv_bn(h, x5, p["bn5d_g"], p["bn5d_b"], pts5, pts4, 4,
                   p["cv4d"], Mt=256)
    h = _ptconv_bn(h, x4, p["bn4d_g"], p["bn4d_b"], pts4, pts3, 4,
                   p["cv3d"], Mt=512)
    h = _ptconv_bn(h, x3, p["bn3d_g"], p["bn3d_b"], pts3, pts2, 8,
                   p["cv2d"], Mt=512)
    h = _ptconv_bn(h, x2, p["bn2d_g"], p["bn2d_b"], pts2, input_pts, 8,
                   p["cv1d"], Mt=256)
    xout, cout = _bn_relu_heads(h, p["bn1d_g"], p["bn1d_b"], p["fcout_w"],
                                p["fcout_b"], p["ccout_w"], p["ccout_b"])
    return (xout, cout)


# speedup vs baseline: 1.5691x; 1.1499x over previous
import functools

import jax
import jax.numpy as jnp
from jax import lax
from jax.experimental import pallas as pl

PL = 48
NC = 16
DIM = 3
OUT_CH = 13


def _ptconv_tile(t, q_ref, out_ref, zcat, pts_t, cen, w1, b1, w2, b2,
                 w3, b3, wm, *, K, N, C, Mt, cout, P):
    q = q_ref[0, pl.ds(t * Mt, Mt), :]

    d2 = None
    for d in range(DIM):
        diff = q[:, d:d + 1] - pts_t[d:d + 1, :]
        sq = diff * diff
        d2 = sq if d2 is None else d2 + sq

    ohs = []
    d2w = d2
    for _ in range(K):
        mn = jnp.min(d2w, axis=1, keepdims=True)
        hot = d2w == mn
        ohs.append(hot.astype(jnp.bfloat16))
        d2w = jnp.where(hot, jnp.float32(jnp.inf), d2w)

    oh = jnp.concatenate(ohs, axis=0)
    g2 = jnp.dot(oh, zcat, preferred_element_type=jnp.float32)
    g = g2[:, :P] + g2[:, P:2 * P]
    fks = [g[k * Mt:(k + 1) * Mt, :C] for k in range(K)]
    prels = [g[k * Mt:(k + 1) * Mt, C:C + DIM] - q for k in range(K)]

    maxr = None
    for pr in prels:
        r2 = jnp.sum(pr * pr, axis=1, keepdims=True)
        maxr = r2 if maxr is None else jnp.maximum(maxr, r2)
    maxi = jnp.sqrt(maxr)
    maxi = jnp.where(maxi == 0.0, 1.0, maxi)
    inv = 1.0 / maxi

    pn_all = jnp.concatenate(prels, axis=0) * jnp.concatenate(
        [inv] * K, axis=0)
    dm = jnp.concatenate(
        [pn_all[:, d:d + 1] - cen[d:d + 1, :] for d in range(DIM)],
        axis=1)
    h = jnp.maximum(
        jnp.dot(dm, w1, preferred_element_type=jnp.float32) + b1, 0.0)
    h = jnp.maximum(
        jnp.dot(h, w2, preferred_element_type=jnp.float32) + b2, 0.0)
    dv = jnp.maximum(
        jnp.dot(h, w3, preferred_element_type=jnp.float32) + b3, 0.0)

    fs_n = [None] * NC
    for k in range(K):
        dvk = dv[k * Mt:(k + 1) * Mt]
        for n in range(NC):
            t2 = fks[k] * dvk[:, n:n + 1]
            fs_n[n] = t2 if fs_n[n] is None else fs_n[n] + t2
    fs = jnp.concatenate(fs_n, axis=1)

    out = jnp.dot(fs, wm, preferred_element_type=jnp.float32)
    out_ref[0, pl.ds(t * Mt, Mt), :] = out * (1.0 / K)


def _split_cat(z, P):
    z_hi = z.astype(jnp.bfloat16)
    z_lo = (z - z_hi.astype(jnp.float32)).astype(jnp.bfloat16)
    pad = P - z.shape[-1]
    zs = jnp.zeros(z.shape[:-1] + (pad,), jnp.bfloat16)
    return jnp.concatenate([z_hi, zs, z_lo, zs], axis=-1)


def _ptconv_body(zcat_ref, ptst_ref, q_ref, cen_ref,
                 w1_ref, b1_ref, w2_ref, b2_ref, w3_ref, b3_ref, wm_ref,
                 out_ref, *, K, N, C, Mt, cout, T, P):
    zcat = zcat_ref[0]
    pts_t = ptst_ref[0]
    args = (zcat, pts_t, cen_ref[...], w1_ref[...], b1_ref[...],
            w2_ref[...], b2_ref[...], w3_ref[...], b3_ref[...], wm_ref[...])

    def tile(t, carry):
        _ptconv_tile(t, q_ref, out_ref, *args, K=K, N=N, C=C, Mt=Mt,
                     cout=cout, P=P)
        return carry

    lax.fori_loop(0, T, tile, 0, unroll=False)


def _ptconv_bn_body(hraw_ref, skip_ref, bg_ref, bb_ref, pts_ref, ptst_ref,
                    q_ref, cen_ref, w1_ref, b1_ref, w2_ref, b2_ref, w3_ref,
                    b3_ref, wm_ref, out_ref, *, K, N, C, Cp, Mt, cout, T, P):
    b = pl.program_id(0)
    hall = hraw_ref[...]
    m = jnp.mean(hall, axis=0, keepdims=True)
    xc = hall - m
    v = jnp.mean(xc * xc, axis=0, keepdims=True)
    hb = hraw_ref[pl.ds(b * N, N), :]
    y = jnp.maximum((hb - m) / jnp.sqrt(v + 1e-5) * bg_ref[...] + bb_ref[...],
                    0.0)
    z = jnp.concatenate([y, skip_ref[0], pts_ref[0]], axis=1)
    zcat = _split_cat(z, P)
    pts_t = ptst_ref[0]
    args = (zcat, pts_t, cen_ref[...], w1_ref[...], b1_ref[...],
            w2_ref[...], b2_ref[...], w3_ref[...], b3_ref[...], wm_ref[...])

    def tile(t, carry):
        _ptconv_tile(t, q_ref, out_ref, *args, K=K, N=N, C=C, Mt=Mt,
                     cout=cout, P=P)
        return carry

    lax.fori_loop(0, T, tile, 0, unroll=False)


def _mlp_args(p):
    return (p["centers"], p["l1w"].T, p["l1b"].reshape(1, -1), p["l2w"].T,
            p["l2b"].reshape(1, -1), p["l3w"].T, p["l3b"].reshape(1, -1))


def _mlp_specs(full2d):
    return [
        full2d((DIM, NC)),
        full2d((DIM * NC, 2 * NC)),
        full2d((1, 2 * NC)),
        full2d((2 * NC, NC)),
        full2d((1, NC)),
        full2d((NC, NC)),
        full2d((1, NC)),
    ]


def _ptconv_first(feat, points, queries, K, p, Mt):
    B, N, C = feat.shape
    M = queries.shape[1]
    cout = p["weight"].shape[2]
    points_t = jnp.transpose(points, (0, 2, 1))
    z = jnp.concatenate([feat, points], axis=2)
    P = 128 * ((C + DIM + 127) // 128)
    zcat = _split_cat(z, P)
    wmat = jnp.transpose(p["weight"], (1, 0, 2)).reshape(NC * C, cout)
    body = functools.partial(_ptconv_body, K=K, N=N, C=C, Mt=Mt, cout=cout,
                             T=M // Mt, P=P)
    full2d = lambda shape: pl.BlockSpec(shape, lambda b: (0, 0))
    out = pl.pallas_call(
        body,
        grid=(B,),
        in_specs=[
            pl.BlockSpec((1, N, 2 * P), lambda b: (b, 0, 0)),
            pl.BlockSpec((1, DIM, N), lambda b: (b, 0, 0)),
            pl.BlockSpec((1, M, DIM), lambda b: (b, 0, 0)),
        ] + _mlp_specs(full2d) + [full2d((NC * C, cout))],
        out_specs=pl.BlockSpec((1, M, cout), lambda b: (b, 0, 0)),
        out_shape=jax.ShapeDtypeStruct((B, M, cout), jnp.float32),
    )(zcat, points_t, queries, *_mlp_args(p), wmat)
    return out


def _ptconv_bn(hraw, skip, bg, bb, points, queries, K, p, Mt):
    B, N, Cp = hraw.shape
    Cs = skip.shape[2]
    C = Cp + Cs
    M = queries.shape[1]
    cout = p["weight"].shape[2]
    points_t = jnp.transpose(points, (0, 2, 1))
    wmat = jnp.transpose(p["weight"], (1, 0, 2)).reshape(NC * C, cout)
    P = 128 * ((C + DIM + 127) // 128)
    body = functools.partial(_ptconv_bn_body, K=K, N=N, C=C, Cp=Cp, Mt=Mt,
                             cout=cout, T=M // Mt, P=P)
    full2d = lambda shape: pl.BlockSpec(shape, lambda b: (0, 0))
    out = pl.pallas_call(
        body,
        grid=(B,),
        in_specs=[
            full2d((B * N, Cp)),
            pl.BlockSpec((1, N, Cs), lambda b: (b, 0, 0)),
            full2d((1, Cp)),
            full2d((1, Cp)),
            pl.BlockSpec((1, N, DIM), lambda b: (b, 0, 0)),
            pl.BlockSpec((1, DIM, N), lambda b: (b, 0, 0)),
            pl.BlockSpec((1, M, DIM), lambda b: (b, 0, 0)),
        ] + _mlp_specs(full2d) + [full2d((NC * C, cout))],
        out_specs=pl.BlockSpec((1, M, cout), lambda b: (b, 0, 0)),
        out_shape=jax.ShapeDtypeStruct((B, M, cout), jnp.float32),
    )(hraw.reshape(B * N, Cp), skip, bg.reshape(1, -1), bb.reshape(1, -1),
      points, points_t, queries, *_mlp_args(p), wmat)
    return out


def _bn_stats_body(x_ref, m_ref, v_ref):
    x = x_ref[...]
    m = jnp.mean(x, axis=0, keepdims=True)
    xc = x - m
    m_ref[...] = m
    v_ref[...] = jnp.mean(xc * xc, axis=0, keepdims=True)


def _heads_body(x_ref, m_ref, v_ref, g_ref, b_ref, fw_ref, fb_ref, cw_ref,
                cb_ref, xout_ref, cout_ref):
    xc = x_ref[...] - m_ref[...]
    y = jnp.maximum(
        xc / jnp.sqrt(v_ref[...] + 1e-5) * g_ref[...] + b_ref[...], 0.0)
    xout_ref[...] = (
        jnp.dot(y, fw_ref[...], preferred_element_type=jnp.float32) +
        fb_ref[...])
    cout_ref[...] = (
        jnp.dot(y, cw_ref[...], preferred_element_type=jnp.float32) +
        cb_ref[...])


def _bn_relu_heads(h, g, b, fw, fb, cw, cb):
    B, M, C = h.shape
    R = B * M
    x = h.reshape(R, C)
    mean, var = pl.pallas_call(
        _bn_stats_body,
        out_shape=(
            jax.ShapeDtypeStruct((1, C), jnp.float32),
            jax.ShapeDtypeStruct((1, C), jnp.float32),
        ),
    )(x)
    Rt = 4096
    full2d = lambda shape: pl.BlockSpec(shape, lambda t: (0, 0))
    xout, cout = pl.pallas_call(
        _heads_body,
        grid=(R // Rt,),
        in_specs=[
            pl.BlockSpec((Rt, C), lambda t: (t, 0)),
            full2d((1, C)),
            full2d((1, C)),
            full2d((1, C)),
            full2d((1, C)),
            full2d((C, OUT_CH)),
            full2d((1, OUT_CH)),
            full2d((C, 1)),
            full2d((1, 1)),
        ],
        out_specs=(
            pl.BlockSpec((Rt, OUT_CH), lambda t: (t, 0)),
            pl.BlockSpec((Rt, 1), lambda t: (t, 0)),
        ),
        out_shape=(
            jax.ShapeDtypeStruct((R, OUT_CH), jnp.float32),
            jax.ShapeDtypeStruct((R, 1), jnp.float32),
        ),
    )(x, mean, var, g.reshape(1, -1), b.reshape(1, -1), fw.T,
      fb.reshape(1, -1), cw.T, cb.reshape(1, -1))
    return xout.reshape(B, M, OUT_CH), cout.reshape(B, M, 1)


def kernel(x, input_pts, x6, pts6, x5, pts5, x4, pts4, x3, pts3, x2, pts2,
           params):
    p = params
    h = _ptconv_first(x6, pts6, pts5, 4, p["cv5d"], Mt=64)
    h = _ptconv_bn(h, x5, p["bn5d_g"], p["bn5d_b"], pts5, pts4, 4,
                   p["cv4d"], Mt=256)
    h = _ptconv_bn(h, x4, p["bn4d_g"], p["bn4d_b"], pts4, pts3, 4,
                   p["cv3d"], Mt=512)
    h = _ptconv_bn(h, x3, p["bn3d_g"], p["bn3d_b"], pts3, pts2, 8,
                   p["cv2d"], Mt=512)
    h = _ptconv_bn(h, x2, p["bn2d_g"], p["bn2d_b"], pts2, input_pts, 8,
                   p["cv1d"], Mt=256)
    xout, cout = _bn_relu_heads(h, p["bn1d_g"], p["bn1d_b"], p["fcout_w"],
                                p["fcout_b"], p["ccout_w"], p["ccout_b"])
    return (xout, cout)
